# baseline (speedup 1.0000x reference)
import functools

import jax
import jax.numpy as jnp
from jax import lax
from jax.experimental import pallas as pl
from jax.experimental.pallas import tpu as pltpu
from jax.experimental.pallas import tpu_sc as plsc

NC = 2
NS = 16
NW = NC * NS
LANES = 16
GSZ = 128
GROUP = 13


def _make_kernel(BF, V, E):
    rows_per_w = BF // NW // GSZ
    n_chunks = rows_per_w // GROUP
    mesh = plsc.VectorSubcoreMesh(core_axis_name="c", subcore_axis_name="s")

    @functools.partial(
        pl.kernel,
        out_type=jax.ShapeDtypeStruct((BF // GSZ, GSZ, E), jnp.float32),
        mesh=mesh,
        scratch_types=[
            pltpu.VMEM((rows_per_w, GSZ), jnp.int32),
            pltpu.VMEM((rows_per_w, GSZ), jnp.int32),
            pltpu.VMEM((GROUP, GSZ, E), jnp.float32),
            pltpu.SemaphoreType.DMA,
        ],
        compiler_params=pltpu.CompilerParams(use_tc_tiling_on_sc=False),
    )
    def k(x_hbm, off_hbm, tab_hbm, out_hbm, xv, offv, rows, sem):
        wid = lax.axis_index("s") * NC + lax.axis_index("c")
        rbase = wid * rows_per_w
        pltpu.sync_copy(x_hbm.at[pl.ds(rbase, rows_per_w)], xv)
        pltpu.sync_copy(off_hbm, offv)

        def add_body(i, carry):
            r = i // (GSZ // LANES)
            col = (i % (GSZ // LANES)) * LANES
            xv[r, pl.ds(col, LANES)] = (
                xv[r, pl.ds(col, LANES)] + offv[r, pl.ds(col, LANES)]
            )
            return carry

        lax.fori_loop(0, rows_per_w * (GSZ // LANES), add_body, 0)

        def chunk_body(cix, carry):
            row0 = cix * GROUP
            copies = [
                pltpu.async_copy(tab_hbm.at[xv.at[row0 + j]], rows.at[j], sem)
                for j in range(GROUP)
            ]
            for cp in copies:
                cp.wait()
            pltpu.sync_copy(rows, out_hbm.at[pl.ds(rbase + row0, GROUP)])
            return carry

        lax.fori_loop(0, n_chunks, chunk_body, 0)

    return k


def kernel(x, table):
    B, F = x.shape
    V, E = table.shape
    field_dim = V // F
    BF = B * F
    x_flat = x.reshape(BF // GSZ, GSZ).astype(jnp.int32)
    per_w = BF // NW
    off = ((jnp.arange(per_w, dtype=jnp.int32) % F) * field_dim).reshape(
        per_w // GSZ, GSZ
    )
    out = _make_kernel(BF, V, E)(x_flat, off, table)
    return out.reshape(B, F, E)


# speedup vs baseline: 6.2486x; 6.2486x over previous
import functools

import jax
import jax.numpy as jnp
from jax import lax
from jax.experimental import pallas as pl
from jax.experimental.pallas import tpu as pltpu
from jax.experimental.pallas import tpu_sc as plsc

NC = 2
NS = 16
NW = NC * NS
LANES = 16


def _make_kernel(B, F, V, E):
    fd = V // F
    n_tasks = F * E
    per_w = n_tasks // NW
    seg_len = ((fd + 127) // 128 + 1) * 128
    mesh = plsc.VectorSubcoreMesh(core_axis_name="c", subcore_axis_name="s")

    @functools.partial(
        pl.kernel,
        out_type=jax.ShapeDtypeStruct((F, E, B), jnp.float32),
        mesh=mesh,
        scratch_types=[
            pltpu.VMEM((seg_len,), jnp.float32),
            pltpu.VMEM((B,), jnp.int32),
            pltpu.VMEM((B,), jnp.float32),
        ],
        compiler_params=pltpu.CompilerParams(needs_layout_passes=False),
    )
    def k(xt_hbm, tabt_hbm, out_hbm, seg, idx, row):
        wid = lax.axis_index("s") * NC + lax.axis_index("c")

        def task_body(j, carry):
            p = wid * per_w + j
            f = p // E
            e = p % E
            seg0 = f * fd
            start = seg0 // 128 * 128
            delta = seg0 - start
            pltpu.sync_copy(tabt_hbm.at[e, pl.ds(start, seg_len)], seg)
            pltpu.sync_copy(xt_hbm.at[f], idx)

            def gather_body(i, c2):
                iv = idx[pl.ds(i * LANES, LANES)] + delta
                row[pl.ds(i * LANES, LANES)] = plsc.load_gather(seg, [iv])
                return c2

            lax.fori_loop(0, B // LANES, gather_body, 0)
            pltpu.sync_copy(row, out_hbm.at[f, e])
            return carry

        lax.fori_loop(0, per_w, task_body, 0)

    return k


def kernel(x, table):
    B, F = x.shape
    V, E = table.shape
    out = _make_kernel(B, F, V, E)(x.T.astype(jnp.int32), table.T)
    return out.transpose(2, 0, 1)


# speedup vs baseline: 8.6843x; 1.3898x over previous
import functools

import jax
import jax.numpy as jnp
from jax import lax
from jax.experimental import pallas as pl
from jax.experimental.pallas import tpu as pltpu
from jax.experimental.pallas import tpu_sc as plsc

NC = 2
NS = 16
LANES = 16


def _make_kernel(B, F, V, E):
    fd = V // F
    seg_len = ((fd + 127) // 128 + 1) * 128
    mesh = plsc.VectorSubcoreMesh(core_axis_name="c", subcore_axis_name="s")

    @functools.partial(
        pl.kernel,
        out_type=jax.ShapeDtypeStruct((F, E, B), jnp.float32),
        mesh=mesh,
        scratch_types=[
            pltpu.VMEM((seg_len,), jnp.float32),
            pltpu.VMEM((seg_len,), jnp.float32),
            pltpu.VMEM((B,), jnp.int32),
            pltpu.VMEM((B,), jnp.float32),
            pltpu.VMEM((B,), jnp.float32),
            pltpu.SemaphoreType.DMA,
            pltpu.SemaphoreType.DMA,
            pltpu.SemaphoreType.DMA,
            pltpu.SemaphoreType.DMA,
        ],
        compiler_params=pltpu.CompilerParams(needs_layout_passes=False),
    )
    def k(xt_hbm, tabt_hbm, out_hbm, seg_a, seg_b, idx, row_a, row_b,
          sg0, sg1, sr0, sr1):
        wid = lax.axis_index("s") * NC + lax.axis_index("c")

        @pl.when(wid < F)
        def _():
            f = wid
            seg0 = f * fd
            start = seg0 // 128 * 128
            delta = seg0 - start
            sgs = (sg0, sg1)
            srs = (sr0, sr1)
            segs = (seg_a, seg_b)
            rows = (row_a, row_b)
            cp_seg = [None] * E
            cp_out = [None] * E
            cp_seg[0] = pltpu.async_copy(
                tabt_hbm.at[0, pl.ds(start, seg_len)], segs[0], sgs[0]
            )
            pltpu.sync_copy(xt_hbm.at[f], idx)
            for e in range(E):
                b = e & 1
                if e + 1 < E:
                    cp_seg[e + 1] = pltpu.async_copy(
                        tabt_hbm.at[e + 1, pl.ds(start, seg_len)],
                        segs[1 - b],
                        sgs[1 - b],
                    )
                cp_seg[e].wait()
                if e >= 2:
                    cp_out[e - 2].wait()

                def gather_body(i, c2, b=b):
                    iv = idx[pl.ds(i * LANES, LANES)] + delta
                    rows[b][pl.ds(i * LANES, LANES)] = plsc.load_gather(
                        segs[b], [iv]
                    )
                    return c2

                lax.fori_loop(0, B // LANES, gather_body, 0)
                cp_out[e] = pltpu.async_copy(rows[b], out_hbm.at[f, e], srs[b])
            cp_out[E - 2].wait()
            cp_out[E - 1].wait()

    return k


def kernel(x, table):
    B, F = x.shape
    V, E = table.shape
    out = _make_kernel(B, F, V, E)(x.T.astype(jnp.int32), table.T)
    return out.transpose(2, 0, 1)


# speedup vs baseline: 14.6419x; 1.6860x over previous
import functools

import jax
import jax.numpy as jnp
from jax import lax
from jax.experimental import pallas as pl
from jax.experimental.pallas import tpu as pltpu
from jax.experimental.pallas import tpu_sc as plsc

NC = 2
NS = 16
LANES = 16


def _make_kernel(B, F, V, E):
    fd = V // F
    seg_len = ((fd + 127) // 128 + 1) * 128
    mesh = plsc.VectorSubcoreMesh(core_axis_name="c", subcore_axis_name="s")

    @functools.partial(
        pl.kernel,
        out_type=jax.ShapeDtypeStruct((F, E, B), jnp.float32),
        mesh=mesh,
        scratch_types=[
            pltpu.VMEM((seg_len,), jnp.float32),
            pltpu.VMEM((seg_len,), jnp.float32),
            pltpu.VMEM((B,), jnp.int32),
            pltpu.VMEM((B,), jnp.float32),
            pltpu.VMEM((B,), jnp.float32),
            pltpu.SemaphoreType.DMA,
            pltpu.SemaphoreType.DMA,
            pltpu.SemaphoreType.DMA,
            pltpu.SemaphoreType.DMA,
        ],
        compiler_params=pltpu.CompilerParams(needs_layout_passes=False),
    )
    def k(xt_hbm, tabt_hbm, out_hbm, seg_a, seg_b, idx, row_a, row_b,
          sg0, sg1, sr0, sr1):
        wid = lax.axis_index("s") * NC + lax.axis_index("c")

        @pl.when(wid < F)
        def _():
            f = wid
            seg0 = f * fd
            start = seg0 // 128 * 128
            delta = seg0 - start
            sgs = (sg0, sg1)
            srs = (sr0, sr1)
            segs = (seg_a, seg_b)
            rows = (row_a, row_b)
            cp_seg = [None] * E
            cp_out = [None] * E
            cp_seg[0] = pltpu.async_copy(
                tabt_hbm.at[0, pl.ds(start, seg_len)], segs[0], sgs[0]
            )
            pltpu.sync_copy(xt_hbm.at[f], idx)
            for e in range(E):
                b = e & 1
                if e + 1 < E:
                    cp_seg[e + 1] = pltpu.async_copy(
                        tabt_hbm.at[e + 1, pl.ds(start, seg_len)],
                        segs[1 - b],
                        sgs[1 - b],
                    )
                cp_seg[e].wait()
                if e >= 2:
                    cp_out[e - 2].wait()

                def gather_body(i, b=b):
                    iv = idx[pl.ds(i * LANES, LANES)] + delta
                    rows[b][pl.ds(i * LANES, LANES)] = plsc.load_gather(
                        segs[b], [iv]
                    )

                plsc.parallel_loop(0, B // LANES, unroll=8)(gather_body)
                cp_out[e] = pltpu.async_copy(rows[b], out_hbm.at[f, e], srs[b])
            cp_out[E - 2].wait()
            cp_out[E - 1].wait()

    return k


def kernel(x, table):
    B, F = x.shape
    V, E = table.shape
    out = _make_kernel(B, F, V, E)(x.T.astype(jnp.int32), table.T)
    return out.transpose(2, 0, 1)
